# baseline (speedup 1.0000x reference)
import jax
import jax.numpy as jnp
from jax import lax
from jax.experimental import pallas as pl
from jax.experimental.pallas import tpu as pltpu
from jax.experimental.pallas import tpu_sc as plsc

B, T, D = 4, 4096, 2048
L = 16
Q = D // 4
NSUB = 16
CHUNKS = Q // L

_DNUMS = lax.GatherDimensionNumbers(
    offset_dims=(), collapsed_slice_dims=(0,), start_index_map=(0,))


def _lane_gather(x, idx):
    return lax.gather(x, idx[:, None], _DNUMS, (1,),
                      mode=lax.GatherScatterMode.PROMISE_IN_BOUNDS)


def _sc_body(lat_hbm, len_hbm, mk_hbm, out_hbm, part_hbm, len_v, idx_v,
             rows_v, mk_v, acc_v, sum_v, out_v, sem, sem2):
    sid = lax.axis_index("s")
    q = sid & 3
    mkcp = pltpu.async_copy(mk_hbm.at[pl.ds(q * Q, Q)], mk_v, sem2)

    pltpu.sync_copy(len_hbm, len_v.at[pl.ds(0, B)])
    lane = lax.iota(jnp.int32, L)
    eos = jnp.maximum(len_v[...] - 1, 1)
    rows = jnp.where(lane < B, lane * T + eos, 0)
    half = jnp.right_shift(lane, 3)
    idx_v[pl.ds(0, L)] = _lane_gather(rows, half)
    idx_v[pl.ds(L, L)] = _lane_gather(rows, half + 2)
    b = jnp.right_shift(sid, 2)
    cp = pltpu.async_copy(lat_hbm.at[idx_v.at[pl.ds(b * 8, 1)]], rows_v, sem)
    mkcp.wait()
    cp.wait()

    acc = jnp.zeros((L,), jnp.float32)
    for c in range(CHUNKS):
        d = rows_v[0, pl.ds(q * Q + c * L, L)] - mk_v[pl.ds(c * L, L)]
        acc = acc + d * d
    acc_v[...] = acc
    pltpu.sync_copy(acc_v, part_hbm.at[sid])

    plsc.subcore_barrier()

    @pl.when(sid == 0)
    def _():
        pltpu.sync_copy(part_hbm, sum_v)
        tot = jnp.zeros((L,), jnp.float32)
        for i in range(NSUB):
            tot = tot + sum_v[i, :]
        for sh in (8, 4, 2, 1):
            tot = tot + _lane_gather(tot, lane ^ sh)
        out_v[...] = tot * (1.0 / float(B * D))
        pltpu.sync_copy(out_v.at[pl.ds(0, 1)], out_hbm)


@jax.jit
def kernel(latents, latent_lengths, marker):
    lat2d = latents.reshape(B * T, D)
    lens = latent_lengths.astype(jnp.int32)

    mesh = plsc.VectorSubcoreMesh(core_axis_name="c", subcore_axis_name="s",
                                  num_cores=1)
    run = pl.kernel(
        _sc_body,
        out_type=(
            jax.ShapeDtypeStruct((1,), jnp.float32),
            jax.ShapeDtypeStruct((NSUB, L), jnp.float32),
        ),
        mesh=mesh,
        scratch_types=[
            pltpu.VMEM((L,), jnp.int32),
            pltpu.VMEM((B * 8,), jnp.int32),
            pltpu.VMEM((1, D), jnp.float32),
            pltpu.VMEM((Q,), jnp.float32),
            pltpu.VMEM((L,), jnp.float32),
            pltpu.VMEM((NSUB, L), jnp.float32),
            pltpu.VMEM((L,), jnp.float32),
            pltpu.SemaphoreType.DMA,
            pltpu.SemaphoreType.DMA,
        ],
    )
    out, _ = run(lat2d, lens, marker)
    return out.reshape(())


# speedup vs baseline: 1.0040x; 1.0040x over previous
import jax
import jax.numpy as jnp
from jax import lax
from jax.experimental import pallas as pl
from jax.experimental.pallas import tpu as pltpu
from jax.experimental.pallas import tpu_sc as plsc

B, T, D = 4, 4096, 2048
L = 16
Q = D // 4
NSUB = 16
CHUNKS = Q // L

_DNUMS = lax.GatherDimensionNumbers(
    offset_dims=(), collapsed_slice_dims=(0,), start_index_map=(0,))


def _lane_gather(x, idx):
    return lax.gather(x, idx[:, None], _DNUMS, (1,),
                      mode=lax.GatherScatterMode.PROMISE_IN_BOUNDS)


def _sc_body(lat_hbm, len_hbm, mk_hbm, out_hbm, part_hbm, len_v, idx_v,
             rows_v, mk_v, acc_v, sum_v, out_v, sem, sem2):
    sid = lax.axis_index("s")
    q = sid & 3
    mkcp = pltpu.async_copy(mk_hbm.at[pl.ds(q * Q, Q)], mk_v, sem2)

    pltpu.sync_copy(len_hbm, len_v.at[pl.ds(0, B)])
    lane = lax.iota(jnp.int32, L)
    eos = jnp.maximum(len_v[...] - 1, 1)
    rows = jnp.where(lane < B, lane * T + eos, 0)
    half = jnp.right_shift(lane, 3)
    idx_v[pl.ds(0, L)] = _lane_gather(rows, half)
    idx_v[pl.ds(L, L)] = _lane_gather(rows, half + 2)
    b = jnp.right_shift(sid, 2)
    cp = pltpu.async_copy(lat_hbm.at[idx_v.at[pl.ds(b * 8, 1)]], rows_v, sem)
    mkcp.wait()
    cp.wait()

    def _step(c, acc):
        d = rows_v[0, pl.ds(q * Q + c * L, L)] - mk_v[pl.ds(c * L, L)]
        return acc + d * d

    acc = lax.fori_loop(0, CHUNKS, _step, jnp.zeros((L,), jnp.float32))
    acc_v[...] = acc
    pltpu.sync_copy(acc_v, part_hbm.at[sid])

    plsc.subcore_barrier()

    @pl.when(sid == 0)
    def _():
        pltpu.sync_copy(part_hbm, sum_v)
        tot = jnp.zeros((L,), jnp.float32)
        for i in range(NSUB):
            tot = tot + sum_v[i, :]
        for sh in (8, 4, 2, 1):
            tot = tot + _lane_gather(tot, lane ^ sh)
        out_v[...] = tot * (1.0 / float(B * D))
        pltpu.sync_copy(out_v.at[pl.ds(0, 1)], out_hbm)


@jax.jit
def kernel(latents, latent_lengths, marker):
    lat2d = latents.reshape(B * T, D)
    lens = latent_lengths.astype(jnp.int32)

    mesh = plsc.VectorSubcoreMesh(core_axis_name="c", subcore_axis_name="s",
                                  num_cores=1)
    run = pl.kernel(
        _sc_body,
        out_type=(
            jax.ShapeDtypeStruct((1,), jnp.float32),
            jax.ShapeDtypeStruct((NSUB, L), jnp.float32),
        ),
        mesh=mesh,
        scratch_types=[
            pltpu.VMEM((L,), jnp.int32),
            pltpu.VMEM((B * 8,), jnp.int32),
            pltpu.VMEM((1, D), jnp.float32),
            pltpu.VMEM((Q,), jnp.float32),
            pltpu.VMEM((L,), jnp.float32),
            pltpu.VMEM((NSUB, L), jnp.float32),
            pltpu.VMEM((L,), jnp.float32),
            pltpu.SemaphoreType.DMA,
            pltpu.SemaphoreType.DMA,
        ],
    )
    out, _ = run(lat2d, lens, marker)
    return out.reshape(())


# speedup vs baseline: 1.0266x; 1.0226x over previous
import jax
import jax.numpy as jnp
from jax import lax
from jax.experimental import pallas as pl
from jax.experimental.pallas import tpu as pltpu
from jax.experimental.pallas import tpu_sc as plsc

B, T, D = 4, 4096, 2048
L = 16
Q = D // 4
NSUB = 16
CHUNKS = Q // L

_DNUMS = lax.GatherDimensionNumbers(
    offset_dims=(), collapsed_slice_dims=(0,), start_index_map=(0,))


def _lane_gather(x, idx):
    return lax.gather(x, idx[:, None], _DNUMS, (1,),
                      mode=lax.GatherScatterMode.PROMISE_IN_BOUNDS)


def _sc_body(lat_hbm, len_hbm, mk_hbm, out_hbm, len_v, idx_v,
             rows_v, mk_v, acc_v, shared, sum_v, out_v, sem, sem2):
    sid = lax.axis_index("s")
    q = sid & 3
    mkcp = pltpu.async_copy(mk_hbm.at[pl.ds(q * Q, Q)], mk_v, sem2)

    pltpu.sync_copy(len_hbm, len_v.at[pl.ds(0, B)])
    lane = lax.iota(jnp.int32, L)
    eos = jnp.maximum(len_v[...] - 1, 1)
    rows = jnp.where(lane < B, lane * T + eos, 0)
    half = jnp.right_shift(lane, 3)
    idx_v[pl.ds(0, L)] = _lane_gather(rows, half)
    idx_v[pl.ds(L, L)] = _lane_gather(rows, half + 2)
    b = jnp.right_shift(sid, 2)
    cp = pltpu.async_copy(lat_hbm.at[idx_v.at[pl.ds(b * 8, 1)]], rows_v, sem)
    mkcp.wait()
    cp.wait()

    def _step(c, acc):
        d = rows_v[0, pl.ds(q * Q + c * L, L)] - mk_v[pl.ds(c * L, L)]
        return acc + d * d

    acc = lax.fori_loop(0, CHUNKS, _step, jnp.zeros((L,), jnp.float32))
    acc_v[...] = acc
    pltpu.sync_copy(acc_v, shared.at[sid, pl.ds(0, L)])

    plsc.subcore_barrier()

    @pl.when(sid == 0)
    def _():
        pltpu.sync_copy(shared, sum_v)
        tot = jnp.zeros((L,), jnp.float32)
        for i in range(NSUB):
            tot = tot + sum_v[i, pl.ds(0, L)]
        for sh in (8, 4, 2, 1):
            tot = tot + _lane_gather(tot, lane ^ sh)
        out_v[...] = tot * (1.0 / float(B * D))
        pltpu.sync_copy(out_v.at[pl.ds(0, 1)], out_hbm)


@jax.jit
def kernel(latents, latent_lengths, marker):
    lat2d = latents.reshape(B * T, D)
    lens = latent_lengths.astype(jnp.int32)

    mesh = plsc.VectorSubcoreMesh(core_axis_name="c", subcore_axis_name="s",
                                  num_cores=1)
    run = pl.kernel(
        _sc_body,
        out_type=jax.ShapeDtypeStruct((1,), jnp.float32),
        mesh=mesh,
        scratch_types=[
            pltpu.VMEM((L,), jnp.int32),
            pltpu.VMEM((B * 8,), jnp.int32),
            pltpu.VMEM((1, D), jnp.float32),
            pltpu.VMEM((Q,), jnp.float32),
            pltpu.VMEM((L,), jnp.float32),
            pltpu.VMEM_SHARED((NSUB, 128), jnp.float32),
            pltpu.VMEM((NSUB, 128), jnp.float32),
            pltpu.VMEM((L,), jnp.float32),
            pltpu.SemaphoreType.DMA,
            pltpu.SemaphoreType.DMA,
        ],
    )
    out = run(lat2d, lens, marker)
    return out.reshape(())


# speedup vs baseline: 1.0303x; 1.0036x over previous
import jax
import jax.numpy as jnp
from jax import lax
from jax.experimental import pallas as pl
from jax.experimental.pallas import tpu as pltpu
from jax.experimental.pallas import tpu_sc as plsc

B, T, D = 4, 4096, 2048
L = 16
Q = D // 4
NSUB = 16
CHUNKS = Q // L

_DNUMS = lax.GatherDimensionNumbers(
    offset_dims=(), collapsed_slice_dims=(0,), start_index_map=(0,))


def _lane_gather(x, idx):
    return lax.gather(x, idx[:, None], _DNUMS, (1,),
                      mode=lax.GatherScatterMode.PROMISE_IN_BOUNDS)


def _sc_body(lat_hbm, len_hbm, mk_hbm, out_hbm, len_v, idx_v,
             rows_v, mk_v, acc_v, shared, sum_v, out_v, sem, sem2):
    sid = lax.axis_index("s")
    q = sid & 3
    mkcp = pltpu.async_copy(mk_hbm.at[pl.ds(q * Q, Q)], mk_v, sem2)

    pltpu.sync_copy(len_hbm, len_v.at[pl.ds(0, B)])
    lane = lax.iota(jnp.int32, L)
    eos = jnp.maximum(len_v[...] - 1, 1)
    rows = jnp.where(lane < B, lane * T + eos, 0)
    half = jnp.right_shift(lane, 3)
    idx_v[pl.ds(0, L)] = _lane_gather(rows, half)
    idx_v[pl.ds(L, L)] = _lane_gather(rows, half + 2)
    b = jnp.right_shift(sid, 2)
    cp = pltpu.async_copy(lat_hbm.at[idx_v.at[pl.ds(b * 8, 1)]], rows_v, sem)
    mkcp.wait()
    cp.wait()

    def _step(c, acc):
        d = rows_v[0, pl.ds(q * Q + c * L, L)] - mk_v[pl.ds(c * L, L)]
        return acc + d * d

    acc = lax.fori_loop(0, CHUNKS, _step, jnp.zeros((L,), jnp.float32))
    for sh in (8, 4, 2, 1):
        acc = acc + _lane_gather(acc, lane ^ sh)
    acc_v[...] = acc
    pltpu.sync_copy(acc_v.at[pl.ds(0, 8)], shared.at[0, pl.ds(sid * 8, 8)])

    plsc.subcore_barrier()

    @pl.when(sid == 0)
    def _():
        pltpu.sync_copy(shared, sum_v)
        tot = jnp.zeros((L,), jnp.float32)
        for k in range(8):
            tot = tot + sum_v[0, pl.ds(k * L, L)]
        tot = tot + _lane_gather(tot, lane ^ 8)
        out_v[...] = tot * (1.0 / float(B * D))
        pltpu.sync_copy(out_v.at[pl.ds(0, 1)], out_hbm)


@jax.jit
def kernel(latents, latent_lengths, marker):
    lat2d = latents.reshape(B * T, D)
    lens = latent_lengths.astype(jnp.int32)

    mesh = plsc.VectorSubcoreMesh(core_axis_name="c", subcore_axis_name="s",
                                  num_cores=1)
    run = pl.kernel(
        _sc_body,
        out_type=jax.ShapeDtypeStruct((1,), jnp.float32),
        mesh=mesh,
        scratch_types=[
            pltpu.VMEM((L,), jnp.int32),
            pltpu.VMEM((B * 8,), jnp.int32),
            pltpu.VMEM((1, D), jnp.float32),
            pltpu.VMEM((Q,), jnp.float32),
            pltpu.VMEM((L,), jnp.float32),
            pltpu.VMEM_SHARED((1, 128), jnp.float32),
            pltpu.VMEM((1, 128), jnp.float32),
            pltpu.VMEM((L,), jnp.float32),
            pltpu.SemaphoreType.DMA,
            pltpu.SemaphoreType.DMA,
        ],
    )
    out = run(lat2d, lens, marker)
    return out.reshape(())
